# baseline (speedup 1.0000x reference)
import functools

import jax
import jax.numpy as jnp
from jax.experimental import pallas as pl
from jax.experimental.pallas import tpu as pltpu

F32 = jnp.float32


def _pool_body(x_ref, a_ref, o_ref):
    o_ref[...] = jnp.dot(x_ref[...], a_ref[...], preferred_element_type=F32)


def _avg_pool(x, Hh, Ww):
    N, C = x.shape[0], x.shape[1]
    HW = Hh * Ww
    xflat = x.reshape(N * C, HW)
    hw = jnp.arange(HW, dtype=jnp.int32)
    A = (jnp.equal(hw[:, None] % Ww, jnp.arange(Ww, dtype=jnp.int32)[None, :])
         .astype(F32)) / float(Hh)
    rows = N * C
    blk = 12800
    grid = rows // blk
    return pl.pallas_call(
        _pool_body,
        grid=(grid,),
        in_specs=[
            pl.BlockSpec((blk, HW), lambda i: (i, 0)),
            pl.BlockSpec((HW, Ww), lambda i: (0, 0)),
        ],
        out_specs=pl.BlockSpec((blk, Ww), lambda i: (i, 0)),
        out_shape=jax.ShapeDtypeStruct((rows, Ww), F32),
    )(xflat, A)


def _main_body(T, Nb, C, H,
               zt_ref, y_ref,
               w1fih, w1fhh, b1f, w1bih, w1bhh, b1b,
               w2fih, w2fhh, b2f, w2bih, w2bhh, b2b,
               lin1f, lin1b, lin1bias, lin2f, lin2b, lin2bias,
               outw, outb, bboxw, bboxb,
               scores_ref, deltas_ref,
               u_ref, hsf_ref, hsb_ref, z2_ref):
    G = 4 * H

    def dir_pass(src_ref, wih_ref, whh_ref, bias_ref, hs_ref, reverse):
        zin = src_ref[...].reshape(T * Nb, C)
        u = jnp.dot(zin, wih_ref[...], preferred_element_type=F32) + bias_ref[...]
        u_ref[...] = u.reshape(T, Nb, G)
        whh = whh_ref[...]
        h = jnp.zeros((Nb, H), F32)
        c = jnp.zeros((Nb, H), F32)
        ts = range(T - 1, -1, -1) if reverse else range(T)
        for t in ts:
            g = u_ref[t] + jnp.dot(h, whh, preferred_element_type=F32)
            gi = jax.nn.sigmoid(g[:, 0 * H:1 * H])
            gf = jax.nn.sigmoid(g[:, 1 * H:2 * H])
            gg = jnp.tanh(g[:, 2 * H:3 * H])
            go = jax.nn.sigmoid(g[:, 3 * H:4 * H])
            c = gf * c + gi * gg
            h = go * jnp.tanh(c)
            hs_ref[t] = h

    dir_pass(zt_ref, w1fih, w1fhh, b1f, hsf_ref, False)
    dir_pass(zt_ref, w1bih, w1bhh, b1b, hsb_ref, True)
    rec1 = (jnp.dot(hsf_ref[...].reshape(T * Nb, H), lin1f[...],
                    preferred_element_type=F32)
            + jnp.dot(hsb_ref[...].reshape(T * Nb, H), lin1b[...],
                      preferred_element_type=F32)
            + lin1bias[...])
    z2_ref[...] = rec1.reshape(T, Nb, H)

    dir_pass(z2_ref, w2fih, w2fhh, b2f, hsf_ref, False)
    dir_pass(z2_ref, w2bih, w2bhh, b2b, hsb_ref, True)
    rec2 = (jnp.dot(hsf_ref[...].reshape(T * Nb, H), lin2f[...],
                    preferred_element_type=F32)
            + jnp.dot(hsb_ref[...].reshape(T * Nb, H), lin2b[...],
                      preferred_element_type=F32)
            + lin2bias[...])

    sc = jnp.dot(rec2, outw[...], preferred_element_type=F32) + outb[...]
    scores_ref[...] = sc.reshape(T, Nb, sc.shape[-1])

    deltas_ref[...] = (jnp.dot(y_ref[...], bboxw[...],
                               preferred_element_type=F32) + bboxb[...])


def kernel(x, y,
           l1_f_Wih, l1_f_Whh, l1_f_bih, l1_f_bhh,
           l1_b_Wih, l1_b_Whh, l1_b_bih, l1_b_bhh,
           l1_lin_W, l1_lin_b,
           l2_f_Wih, l2_f_Whh, l2_f_bih, l2_f_bhh,
           l2_b_Wih, l2_b_Whh, l2_b_bih, l2_b_bhh,
           l2_lin_W, l2_lin_b,
           out_W, out_b, bbox_W, bbox_b):
    N, C, Hh, Ww = x.shape
    T = Ww
    H = l1_f_Whh.shape[1]
    G = 4 * H
    NC = out_W.shape[0]
    FC = y.shape[1]
    Nb = 200
    grid = N // Nb

    m = _avg_pool(x, Hh, Ww)
    zt = m.reshape(N, C, T).transpose(2, 0, 1)

    def prep(wih, whh, bih, bhh):
        return wih.T, whh.T, (bih + bhh).reshape(1, G)

    w1fih, w1fhh, b1f = prep(l1_f_Wih, l1_f_Whh, l1_f_bih, l1_f_bhh)
    w1bih, w1bhh, b1b = prep(l1_b_Wih, l1_b_Whh, l1_b_bih, l1_b_bhh)
    w2fih, w2fhh, b2f = prep(l2_f_Wih, l2_f_Whh, l2_f_bih, l2_f_bhh)
    w2bih, w2bhh, b2b = prep(l2_b_Wih, l2_b_Whh, l2_b_bih, l2_b_bhh)
    lin1f = l1_lin_W[:, :H].T
    lin1b = l1_lin_W[:, H:].T
    lin2f = l2_lin_W[:, :H].T
    lin2b = l2_lin_W[:, H:].T
    lin1bias = l1_lin_b.reshape(1, H)
    lin2bias = l2_lin_b.reshape(1, H)
    outw = out_W.T
    outb = out_b.reshape(1, NC)
    bboxw = bbox_W.T
    bboxb = bbox_b.reshape(1, 4)

    full = lambda shape: pl.BlockSpec(shape, lambda i: tuple(0 for _ in shape))
    in_specs = [
        pl.BlockSpec((T, Nb, C), lambda i: (0, i, 0)),
        pl.BlockSpec((Nb, FC), lambda i: (i, 0)),
        full((C, G)), full((H, G)), full((1, G)),
        full((C, G)), full((H, G)), full((1, G)),
        full((H, G)), full((H, G)), full((1, G)),
        full((H, G)), full((H, G)), full((1, G)),
        full((H, H)), full((H, H)), full((1, H)),
        full((H, H)), full((H, H)), full((1, H)),
        full((H, NC)), full((1, NC)),
        full((FC, 4)), full((1, 4)),
    ]
    out_specs = [
        pl.BlockSpec((T, Nb, NC), lambda i: (0, i, 0)),
        pl.BlockSpec((Nb, 4), lambda i: (i, 0)),
    ]
    out_shape = [
        jax.ShapeDtypeStruct((T, N, NC), F32),
        jax.ShapeDtypeStruct((N, 4), F32),
    ]
    scratch_shapes = [
        pltpu.VMEM((T, Nb, G), F32),
        pltpu.VMEM((T, Nb, H), F32),
        pltpu.VMEM((T, Nb, H), F32),
        pltpu.VMEM((T, Nb, H), F32),
    ]

    scores_tm, deltas = pl.pallas_call(
        functools.partial(_main_body, T, Nb, C, H),
        grid=(grid,),
        in_specs=in_specs,
        out_specs=out_specs,
        out_shape=out_shape,
        scratch_shapes=scratch_shapes,
    )(zt, y,
      w1fih, w1fhh, b1f, w1bih, w1bhh, b1b,
      w2fih, w2fhh, b2f, w2bih, w2bhh, b2b,
      lin1f, lin1b, lin1bias, lin2f, lin2b, lin2bias,
      outw, outb, bboxw, bboxb)

    scores = scores_tm.transpose(1, 0, 2)
    return (scores, deltas)


# speedup vs baseline: 1.0585x; 1.0585x over previous
import functools

import jax
import jax.numpy as jnp
from jax.experimental import pallas as pl
from jax.experimental.pallas import tpu as pltpu

F32 = jnp.float32
BF16 = jnp.bfloat16


def _pool_body(x_ref, a_ref, o_ref):
    m = jnp.dot(x_ref[...], a_ref[...], preferred_element_type=F32)
    o_ref[...] = m.astype(BF16)


def _avg_pool(x, Hh, Ww):
    N, C = x.shape[0], x.shape[1]
    HW = Hh * Ww
    xflat = x.reshape(N * C, HW)
    hw = jnp.arange(HW, dtype=jnp.int32)
    A = (jnp.equal(hw[:, None] % Ww, jnp.arange(Ww, dtype=jnp.int32)[None, :])
         .astype(F32)) / float(Hh)
    rows = N * C
    blk = 12800
    grid = rows // blk
    return pl.pallas_call(
        _pool_body,
        grid=(grid,),
        in_specs=[
            pl.BlockSpec((blk, HW), lambda i: (i, 0)),
            pl.BlockSpec((HW, Ww), lambda i: (0, 0)),
        ],
        out_specs=pl.BlockSpec((blk, Ww), lambda i: (i, 0)),
        out_shape=jax.ShapeDtypeStruct((rows, Ww), BF16),
    )(xflat, A)


def _sig(x):
    return 0.5 * jnp.tanh(0.5 * x) + 0.5


def _main_body(T, Nb, C, H,
               zt_ref, y_ref,
               w1fih, w1fhh, b1f, w1bih, w1bhh, b1b,
               w2fih, w2fhh, b2f, w2bih, w2bhh, b2b,
               lin1f, lin1b, lin1bias, lin2f, lin2b, lin2bias,
               outw, outb, bboxw, bboxb,
               scores_ref, deltas_ref,
               uf_ref, ub_ref, hsf_ref, hsb_ref, z2_ref):
    G = 4 * H

    def bilstm(src, wfih, wfhh, bf, wbih, wbhh, bb):
        uf = jnp.dot(src, wfih[...], preferred_element_type=F32) + bf[...]
        uf_ref[...] = uf.reshape(T, Nb, G)
        ub = jnp.dot(src, wbih[...], preferred_element_type=F32) + bb[...]
        ub_ref[...] = ub.reshape(T, Nb, G)
        whf = wfhh[...]
        whb = wbhh[...]
        hf = jnp.zeros((Nb, H), F32)
        cf = jnp.zeros((Nb, H), F32)
        hb = jnp.zeros((Nb, H), F32)
        cb = jnp.zeros((Nb, H), F32)
        for k in range(T):
            tb = T - 1 - k
            gf = uf_ref[k] + jnp.dot(hf, whf, preferred_element_type=F32)
            gb = ub_ref[tb] + jnp.dot(hb, whb, preferred_element_type=F32)
            cf = (_sig(gf[:, 1 * H:2 * H]) * cf
                  + _sig(gf[:, 0 * H:1 * H]) * jnp.tanh(gf[:, 2 * H:3 * H]))
            cb = (_sig(gb[:, 1 * H:2 * H]) * cb
                  + _sig(gb[:, 0 * H:1 * H]) * jnp.tanh(gb[:, 2 * H:3 * H]))
            hf = _sig(gf[:, 3 * H:4 * H]) * jnp.tanh(cf)
            hb = _sig(gb[:, 3 * H:4 * H]) * jnp.tanh(cb)
            hsf_ref[k] = hf
            hsb_ref[tb] = hb

    def lin(wf, wb, bias):
        return (jnp.dot(hsf_ref[...].reshape(T * Nb, H), wf[...],
                        preferred_element_type=F32)
                + jnp.dot(hsb_ref[...].reshape(T * Nb, H), wb[...],
                          preferred_element_type=F32)
                + bias[...])

    z1 = zt_ref[...].astype(F32).reshape(T * Nb, C)
    bilstm(z1, w1fih, w1fhh, b1f, w1bih, w1bhh, b1b)
    rec1 = lin(lin1f, lin1b, lin1bias)
    z2_ref[...] = rec1.reshape(T, Nb, H)

    bilstm(z2_ref[...].reshape(T * Nb, H), w2fih, w2fhh, b2f, w2bih, w2bhh, b2b)
    rec2 = lin(lin2f, lin2b, lin2bias)

    sc = jnp.dot(rec2, outw[...], preferred_element_type=F32) + outb[...]
    sc3 = sc.reshape(T, Nb, sc.shape[-1])
    for t in range(T):
        scores_ref[:, t, :] = sc3[t]

    deltas_ref[...] = (jnp.dot(y_ref[...], bboxw[...],
                               preferred_element_type=F32) + bboxb[...])


def kernel(x, y,
           l1_f_Wih, l1_f_Whh, l1_f_bih, l1_f_bhh,
           l1_b_Wih, l1_b_Whh, l1_b_bih, l1_b_bhh,
           l1_lin_W, l1_lin_b,
           l2_f_Wih, l2_f_Whh, l2_f_bih, l2_f_bhh,
           l2_b_Wih, l2_b_Whh, l2_b_bih, l2_b_bhh,
           l2_lin_W, l2_lin_b,
           out_W, out_b, bbox_W, bbox_b):
    N, C, Hh, Ww = x.shape
    T = Ww
    H = l1_f_Whh.shape[1]
    G = 4 * H
    NC = out_W.shape[0]
    FC = y.shape[1]
    Nb = 200
    grid = N // Nb

    m = _avg_pool(x, Hh, Ww)
    zt = m.reshape(N, C, T).transpose(2, 0, 1)

    def prep(wih, whh, bih, bhh):
        return wih.T, whh.T, (bih + bhh).reshape(1, G)

    w1fih, w1fhh, b1f = prep(l1_f_Wih, l1_f_Whh, l1_f_bih, l1_f_bhh)
    w1bih, w1bhh, b1b = prep(l1_b_Wih, l1_b_Whh, l1_b_bih, l1_b_bhh)
    w2fih, w2fhh, b2f = prep(l2_f_Wih, l2_f_Whh, l2_f_bih, l2_f_bhh)
    w2bih, w2bhh, b2b = prep(l2_b_Wih, l2_b_Whh, l2_b_bih, l2_b_bhh)
    lin1f = l1_lin_W[:, :H].T
    lin1b = l1_lin_W[:, H:].T
    lin2f = l2_lin_W[:, :H].T
    lin2b = l2_lin_W[:, H:].T
    lin1bias = l1_lin_b.reshape(1, H)
    lin2bias = l2_lin_b.reshape(1, H)
    outw = out_W.T
    outb = out_b.reshape(1, NC)
    bboxw = bbox_W.T
    bboxb = bbox_b.reshape(1, 4)

    full = lambda shape: pl.BlockSpec(shape, lambda i: tuple(0 for _ in shape))
    in_specs = [
        pl.BlockSpec((T, Nb, C), lambda i: (0, i, 0)),
        pl.BlockSpec((Nb, FC), lambda i: (i, 0)),
        full((C, G)), full((H, G)), full((1, G)),
        full((C, G)), full((H, G)), full((1, G)),
        full((H, G)), full((H, G)), full((1, G)),
        full((H, G)), full((H, G)), full((1, G)),
        full((H, H)), full((H, H)), full((1, H)),
        full((H, H)), full((H, H)), full((1, H)),
        full((H, NC)), full((1, NC)),
        full((FC, 4)), full((1, 4)),
    ]
    out_specs = [
        pl.BlockSpec((Nb, T, NC), lambda i: (i, 0, 0)),
        pl.BlockSpec((Nb, 4), lambda i: (i, 0)),
    ]
    out_shape = [
        jax.ShapeDtypeStruct((N, T, NC), F32),
        jax.ShapeDtypeStruct((N, 4), F32),
    ]
    scratch_shapes = [
        pltpu.VMEM((T, Nb, G), F32),
        pltpu.VMEM((T, Nb, G), F32),
        pltpu.VMEM((T, Nb, H), F32),
        pltpu.VMEM((T, Nb, H), F32),
        pltpu.VMEM((T, Nb, H), F32),
    ]

    scores, deltas = pl.pallas_call(
        functools.partial(_main_body, T, Nb, C, H),
        grid=(grid,),
        in_specs=in_specs,
        out_specs=out_specs,
        out_shape=out_shape,
        scratch_shapes=scratch_shapes,
    )(zt, y,
      w1fih, w1fhh, b1f, w1bih, w1bhh, b1b,
      w2fih, w2fhh, b2f, w2bih, w2bhh, b2b,
      lin1f, lin1b, lin1bias, lin2f, lin2b, lin2bias,
      outw, outb, bboxw, bboxb)

    return (scores, deltas)


# speedup vs baseline: 1.0672x; 1.0082x over previous
import functools

import jax
import jax.numpy as jnp
from jax.experimental import pallas as pl
from jax.experimental.pallas import tpu as pltpu

F32 = jnp.float32
BF16 = jnp.bfloat16


def _pool_body(x_ref, a_ref, o_ref):
    m = jnp.dot(x_ref[...], a_ref[...], preferred_element_type=F32)
    o_ref[...] = m.astype(BF16)


def _avg_pool(x, Hh, Ww):
    N, C = x.shape[0], x.shape[1]
    HW = Hh * Ww
    xflat = x.reshape(N * C, HW)
    hw = jnp.arange(HW, dtype=jnp.int32)
    A = (jnp.equal(hw[:, None] % Ww, jnp.arange(Ww, dtype=jnp.int32)[None, :])
         .astype(F32)) / float(Hh)
    rows = N * C
    blk = 12800
    grid = rows // blk
    return pl.pallas_call(
        _pool_body,
        grid=(grid,),
        in_specs=[
            pl.BlockSpec((blk, HW), lambda i: (i, 0)),
            pl.BlockSpec((HW, Ww), lambda i: (0, 0)),
        ],
        out_specs=pl.BlockSpec((blk, Ww), lambda i: (i, 0)),
        out_shape=jax.ShapeDtypeStruct((rows, Ww), BF16),
    )(xflat, A)


def _sig(x):
    return 0.5 * jnp.tanh(0.5 * x) + 0.5


def _main_body(T, Nb, C, H,
               zt_ref, y_ref,
               w1fih, w1fhh, b1f, w1bih, w1bhh, b1b,
               w2fih, w2fhh, b2f, w2bih, w2bhh, b2b,
               lin1f, lin1b, lin1bias, lin2f, lin2b, lin2bias,
               outw, outb, bboxw, bboxb,
               scores_ref, deltas_ref,
               uf_ref, ub_ref, hsf_ref, hsb_ref, z2_ref):
    G = 4 * H

    def bilstm(src, wfih, wfhh, bf, wbih, wbhh, bb):
        uf = jnp.dot(src, wfih[...], preferred_element_type=F32) + bf[...]
        uf_ref[...] = uf.reshape(T, Nb, G)
        ub = jnp.dot(src, wbih[...], preferred_element_type=F32) + bb[...]
        ub_ref[...] = ub.reshape(T, Nb, G)
        whf = wfhh[...]
        whb = wbhh[...]
        hf = jnp.zeros((Nb, H), BF16)
        cf = jnp.zeros((Nb, H), F32)
        hb = jnp.zeros((Nb, H), BF16)
        cb = jnp.zeros((Nb, H), F32)
        for k in range(T):
            tb = T - 1 - k
            gf = uf_ref[k] + jnp.dot(hf, whf, preferred_element_type=F32)
            gb = ub_ref[tb] + jnp.dot(hb, whb, preferred_element_type=F32)
            cf = (_sig(gf[:, 1 * H:2 * H]) * cf
                  + _sig(gf[:, 0 * H:1 * H]) * jnp.tanh(gf[:, 2 * H:3 * H]))
            cb = (_sig(gb[:, 1 * H:2 * H]) * cb
                  + _sig(gb[:, 0 * H:1 * H]) * jnp.tanh(gb[:, 2 * H:3 * H]))
            hf = (_sig(gf[:, 3 * H:4 * H]) * jnp.tanh(cf)).astype(BF16)
            hb = (_sig(gb[:, 3 * H:4 * H]) * jnp.tanh(cb)).astype(BF16)
            hsf_ref[k] = hf
            hsb_ref[tb] = hb

    def lin(wf, wb, bias):
        return (jnp.dot(hsf_ref[...].reshape(T * Nb, H), wf[...],
                        preferred_element_type=F32)
                + jnp.dot(hsb_ref[...].reshape(T * Nb, H), wb[...],
                          preferred_element_type=F32)
                + bias[...])

    z1 = zt_ref[...].reshape(T * Nb, C)
    bilstm(z1, w1fih, w1fhh, b1f, w1bih, w1bhh, b1b)
    rec1 = lin(lin1f, lin1b, lin1bias)
    z2_ref[...] = rec1.astype(BF16).reshape(T, Nb, H)

    bilstm(z2_ref[...].reshape(T * Nb, H), w2fih, w2fhh, b2f, w2bih, w2bhh, b2b)
    rec2 = lin(lin2f, lin2b, lin2bias)

    sc = jnp.dot(rec2.astype(BF16), outw[...], preferred_element_type=F32) + outb[...]
    sc3 = sc.reshape(T, Nb, sc.shape[-1])
    for t in range(T):
        scores_ref[:, t, :] = sc3[t]

    deltas_ref[...] = (jnp.dot(y_ref[...].astype(BF16), bboxw[...],
                               preferred_element_type=F32) + bboxb[...])


def kernel(x, y,
           l1_f_Wih, l1_f_Whh, l1_f_bih, l1_f_bhh,
           l1_b_Wih, l1_b_Whh, l1_b_bih, l1_b_bhh,
           l1_lin_W, l1_lin_b,
           l2_f_Wih, l2_f_Whh, l2_f_bih, l2_f_bhh,
           l2_b_Wih, l2_b_Whh, l2_b_bih, l2_b_bhh,
           l2_lin_W, l2_lin_b,
           out_W, out_b, bbox_W, bbox_b):
    N, C, Hh, Ww = x.shape
    T = Ww
    H = l1_f_Whh.shape[1]
    G = 4 * H
    NC = out_W.shape[0]
    FC = y.shape[1]
    Nb = 200
    grid = N // Nb

    m = _avg_pool(x, Hh, Ww)
    zt = m.reshape(N, C, T).transpose(2, 0, 1)

    def prep(wih, whh, bih, bhh):
        return (wih.T.astype(BF16), whh.T.astype(BF16),
                (bih + bhh).reshape(1, G))

    w1fih, w1fhh, b1f = prep(l1_f_Wih, l1_f_Whh, l1_f_bih, l1_f_bhh)
    w1bih, w1bhh, b1b = prep(l1_b_Wih, l1_b_Whh, l1_b_bih, l1_b_bhh)
    w2fih, w2fhh, b2f = prep(l2_f_Wih, l2_f_Whh, l2_f_bih, l2_f_bhh)
    w2bih, w2bhh, b2b = prep(l2_b_Wih, l2_b_Whh, l2_b_bih, l2_b_bhh)
    lin1f = l1_lin_W[:, :H].T.astype(BF16)
    lin1b = l1_lin_W[:, H:].T.astype(BF16)
    lin2f = l2_lin_W[:, :H].T.astype(BF16)
    lin2b = l2_lin_W[:, H:].T.astype(BF16)
    lin1bias = l1_lin_b.reshape(1, H)
    lin2bias = l2_lin_b.reshape(1, H)
    outw = out_W.T.astype(BF16)
    outb = out_b.reshape(1, NC)
    bboxw = bbox_W.T.astype(BF16)
    bboxb = bbox_b.reshape(1, 4)

    full = lambda shape: pl.BlockSpec(shape, lambda i: tuple(0 for _ in shape))
    in_specs = [
        pl.BlockSpec((T, Nb, C), lambda i: (0, i, 0)),
        pl.BlockSpec((Nb, FC), lambda i: (i, 0)),
        full((C, G)), full((H, G)), full((1, G)),
        full((C, G)), full((H, G)), full((1, G)),
        full((H, G)), full((H, G)), full((1, G)),
        full((H, G)), full((H, G)), full((1, G)),
        full((H, H)), full((H, H)), full((1, H)),
        full((H, H)), full((H, H)), full((1, H)),
        full((H, NC)), full((1, NC)),
        full((FC, 4)), full((1, 4)),
    ]
    out_specs = [
        pl.BlockSpec((Nb, T, NC), lambda i: (i, 0, 0)),
        pl.BlockSpec((Nb, 4), lambda i: (i, 0)),
    ]
    out_shape = [
        jax.ShapeDtypeStruct((N, T, NC), F32),
        jax.ShapeDtypeStruct((N, 4), F32),
    ]
    scratch_shapes = [
        pltpu.VMEM((T, Nb, G), F32),
        pltpu.VMEM((T, Nb, G), F32),
        pltpu.VMEM((T, Nb, H), BF16),
        pltpu.VMEM((T, Nb, H), BF16),
        pltpu.VMEM((T, Nb, H), BF16),
    ]

    scores, deltas = pl.pallas_call(
        functools.partial(_main_body, T, Nb, C, H),
        grid=(grid,),
        in_specs=in_specs,
        out_specs=out_specs,
        out_shape=out_shape,
        scratch_shapes=scratch_shapes,
    )(zt, y,
      w1fih, w1fhh, b1f, w1bih, w1bhh, b1b,
      w2fih, w2fhh, b2f, w2bih, w2bhh, b2b,
      lin1f, lin1b, lin1bias, lin2f, lin2b, lin2bias,
      outw, outb, bboxw, bboxb)

    return (scores, deltas)


# speedup vs baseline: 1.5640x; 1.4654x over previous
import functools

import jax
import jax.numpy as jnp
from jax.experimental import pallas as pl
from jax.experimental.pallas import tpu as pltpu

F32 = jnp.float32
BF16 = jnp.bfloat16


def _pool_body(C, x_ref, a_ref, o_ref):
    nb = x_ref.shape[0]
    xf = x_ref[...].reshape(nb * C, x_ref.shape[2])
    m = jnp.dot(xf, a_ref[...], preferred_element_type=F32)
    o_ref[...] = m.astype(BF16)


def _avg_pool(x, Hh, Ww):
    N, C = x.shape[0], x.shape[1]
    HW = Hh * Ww
    x3 = x.reshape(N, C, HW)
    hw = jnp.arange(HW, dtype=jnp.int32)
    A = (jnp.equal(hw[:, None] % Ww, jnp.arange(Ww, dtype=jnp.int32)[None, :])
         .astype(F32)) / float(Hh)
    nb = 50
    grid = N // nb
    return pl.pallas_call(
        functools.partial(_pool_body, C),
        grid=(grid,),
        in_specs=[
            pl.BlockSpec((nb, C, HW), lambda i: (i, 0, 0)),
            pl.BlockSpec((HW, Ww), lambda i: (0, 0)),
        ],
        out_specs=pl.BlockSpec((nb * C, Ww), lambda i: (i, 0)),
        out_shape=jax.ShapeDtypeStruct((N * C, Ww), BF16),
    )(x3, A)


def _sig(x):
    return 0.5 * jnp.tanh(0.5 * x) + 0.5


def _main_body(T, Nb, C, H,
               zt_ref, y_ref,
               w1fih, w1fhh, b1f, w1bih, w1bhh, b1b,
               w2fih, w2fhh, b2f, w2bih, w2bhh, b2b,
               lin1f, lin1b, lin1bias, lin2f, lin2b, lin2bias,
               outw, outb, bboxw, bboxb,
               scores_ref, deltas_ref,
               uf_ref, ub_ref, hsf_ref, hsb_ref, z2_ref):
    G = 4 * H

    def bilstm(src, wfih, wfhh, bf, wbih, wbhh, bb):
        uf = jnp.dot(src, wfih[...], preferred_element_type=F32) + bf[...]
        uf_ref[...] = uf.reshape(T, Nb, G)
        ub = jnp.dot(src, wbih[...], preferred_element_type=F32) + bb[...]
        ub_ref[...] = ub.reshape(T, Nb, G)
        whf = wfhh[...]
        whb = wbhh[...]
        hf = jnp.zeros((Nb, H), BF16)
        cf = jnp.zeros((Nb, H), F32)
        hb = jnp.zeros((Nb, H), BF16)
        cb = jnp.zeros((Nb, H), F32)
        for k in range(T):
            tb = T - 1 - k
            gf = uf_ref[k] + jnp.dot(hf, whf, preferred_element_type=F32)
            gb = ub_ref[tb] + jnp.dot(hb, whb, preferred_element_type=F32)
            cf = (_sig(gf[:, 1 * H:2 * H]) * cf
                  + _sig(gf[:, 0 * H:1 * H]) * jnp.tanh(gf[:, 2 * H:3 * H]))
            cb = (_sig(gb[:, 1 * H:2 * H]) * cb
                  + _sig(gb[:, 0 * H:1 * H]) * jnp.tanh(gb[:, 2 * H:3 * H]))
            hf = (_sig(gf[:, 3 * H:4 * H]) * jnp.tanh(cf)).astype(BF16)
            hb = (_sig(gb[:, 3 * H:4 * H]) * jnp.tanh(cb)).astype(BF16)
            hsf_ref[k] = hf
            hsb_ref[tb] = hb

    def lin(wf, wb, bias):
        return (jnp.dot(hsf_ref[...].reshape(T * Nb, H), wf[...],
                        preferred_element_type=F32)
                + jnp.dot(hsb_ref[...].reshape(T * Nb, H), wb[...],
                          preferred_element_type=F32)
                + bias[...])

    z1 = zt_ref[...].reshape(T * Nb, C)
    bilstm(z1, w1fih, w1fhh, b1f, w1bih, w1bhh, b1b)
    rec1 = lin(lin1f, lin1b, lin1bias)
    z2_ref[...] = rec1.astype(BF16).reshape(T, Nb, H)

    bilstm(z2_ref[...].reshape(T * Nb, H), w2fih, w2fhh, b2f, w2bih, w2bhh, b2b)
    rec2 = lin(lin2f, lin2b, lin2bias)

    sc = jnp.dot(rec2.astype(BF16), outw[...], preferred_element_type=F32) + outb[...]
    sc3 = sc.reshape(T, Nb, sc.shape[-1])
    for t in range(T):
        scores_ref[:, t, :] = sc3[t]

    deltas_ref[...] = (jnp.dot(y_ref[...].astype(BF16), bboxw[...],
                               preferred_element_type=F32) + bboxb[...])


def kernel(x, y,
           l1_f_Wih, l1_f_Whh, l1_f_bih, l1_f_bhh,
           l1_b_Wih, l1_b_Whh, l1_b_bih, l1_b_bhh,
           l1_lin_W, l1_lin_b,
           l2_f_Wih, l2_f_Whh, l2_f_bih, l2_f_bhh,
           l2_b_Wih, l2_b_Whh, l2_b_bih, l2_b_bhh,
           l2_lin_W, l2_lin_b,
           out_W, out_b, bbox_W, bbox_b):
    N, C, Hh, Ww = x.shape
    T = Ww
    H = l1_f_Whh.shape[1]
    G = 4 * H
    NC = out_W.shape[0]
    FC = y.shape[1]
    Nb = 200
    grid = N // Nb

    m = _avg_pool(x, Hh, Ww)
    zt = m.reshape(N, C, T).transpose(2, 0, 1)

    def prep(wih, whh, bih, bhh):
        return (wih.T.astype(BF16), whh.T.astype(BF16),
                (bih + bhh).reshape(1, G))

    w1fih, w1fhh, b1f = prep(l1_f_Wih, l1_f_Whh, l1_f_bih, l1_f_bhh)
    w1bih, w1bhh, b1b = prep(l1_b_Wih, l1_b_Whh, l1_b_bih, l1_b_bhh)
    w2fih, w2fhh, b2f = prep(l2_f_Wih, l2_f_Whh, l2_f_bih, l2_f_bhh)
    w2bih, w2bhh, b2b = prep(l2_b_Wih, l2_b_Whh, l2_b_bih, l2_b_bhh)
    lin1f = l1_lin_W[:, :H].T.astype(BF16)
    lin1b = l1_lin_W[:, H:].T.astype(BF16)
    lin2f = l2_lin_W[:, :H].T.astype(BF16)
    lin2b = l2_lin_W[:, H:].T.astype(BF16)
    lin1bias = l1_lin_b.reshape(1, H)
    lin2bias = l2_lin_b.reshape(1, H)
    outw = out_W.T.astype(BF16)
    outb = out_b.reshape(1, NC)
    bboxw = bbox_W.T.astype(BF16)
    bboxb = bbox_b.reshape(1, 4)

    full = lambda shape: pl.BlockSpec(shape, lambda i: tuple(0 for _ in shape))
    in_specs = [
        pl.BlockSpec((T, Nb, C), lambda i: (0, i, 0)),
        pl.BlockSpec((Nb, FC), lambda i: (i, 0)),
        full((C, G)), full((H, G)), full((1, G)),
        full((C, G)), full((H, G)), full((1, G)),
        full((H, G)), full((H, G)), full((1, G)),
        full((H, G)), full((H, G)), full((1, G)),
        full((H, H)), full((H, H)), full((1, H)),
        full((H, H)), full((H, H)), full((1, H)),
        full((H, NC)), full((1, NC)),
        full((FC, 4)), full((1, 4)),
    ]
    out_specs = [
        pl.BlockSpec((Nb, T, NC), lambda i: (i, 0, 0)),
        pl.BlockSpec((Nb, 4), lambda i: (i, 0)),
    ]
    out_shape = [
        jax.ShapeDtypeStruct((N, T, NC), F32),
        jax.ShapeDtypeStruct((N, 4), F32),
    ]
    scratch_shapes = [
        pltpu.VMEM((T, Nb, G), F32),
        pltpu.VMEM((T, Nb, G), F32),
        pltpu.VMEM((T, Nb, H), BF16),
        pltpu.VMEM((T, Nb, H), BF16),
        pltpu.VMEM((T, Nb, H), BF16),
    ]

    scores, deltas = pl.pallas_call(
        functools.partial(_main_body, T, Nb, C, H),
        grid=(grid,),
        in_specs=in_specs,
        out_specs=out_specs,
        out_shape=out_shape,
        scratch_shapes=scratch_shapes,
    )(zt, y,
      w1fih, w1fhh, b1f, w1bih, w1bhh, b1b,
      w2fih, w2fhh, b2f, w2bih, w2bhh, b2b,
      lin1f, lin1b, lin1bias, lin2f, lin2b, lin2bias,
      outw, outb, bboxw, bboxb)

    return (scores, deltas)


# speedup vs baseline: 3.7254x; 2.3820x over previous
import functools

import jax
import jax.numpy as jnp
from jax.experimental import pallas as pl
from jax.experimental.pallas import tpu as pltpu

F32 = jnp.float32
BF16 = jnp.bfloat16


def _pool_body(inv_h, x_ref, o_ref):
    s = jnp.sum(x_ref[...], axis=0)
    o_ref[...] = (s * inv_h).astype(BF16)


def _avg_pool(x, Hh, Ww):
    N, C = x.shape[0], x.shape[1]
    xt = jnp.transpose(x, (2, 3, 0, 1))
    nb = 40
    grid = N // nb
    return pl.pallas_call(
        functools.partial(_pool_body, 1.0 / float(Hh)),
        grid=(grid,),
        in_specs=[
            pl.BlockSpec((Hh, Ww, nb, C), lambda i: (0, 0, i, 0)),
        ],
        out_specs=pl.BlockSpec((Ww, nb, C), lambda i: (0, i, 0)),
        out_shape=jax.ShapeDtypeStruct((Ww, N, C), BF16),
    )(xt)


def _sig(x):
    return 0.5 * jnp.tanh(0.5 * x) + 0.5


def _main_body(T, Nb, C, H,
               zt_ref, y_ref,
               w1fih, w1fhh, b1f, w1bih, w1bhh, b1b,
               w2fih, w2fhh, b2f, w2bih, w2bhh, b2b,
               lin1f, lin1b, lin1bias, lin2f, lin2b, lin2bias,
               outw, outb, bboxw, bboxb,
               scores_ref, deltas_ref,
               uf_ref, ub_ref, hsf_ref, hsb_ref, z2_ref):
    G = 4 * H

    def bilstm(src, wfih, wfhh, bf, wbih, wbhh, bb):
        uf = jnp.dot(src, wfih[...], preferred_element_type=F32) + bf[...]
        uf_ref[...] = uf.reshape(T, Nb, G)
        ub = jnp.dot(src, wbih[...], preferred_element_type=F32) + bb[...]
        ub_ref[...] = ub.reshape(T, Nb, G)
        whf = wfhh[...]
        whb = wbhh[...]
        hf = jnp.zeros((Nb, H), BF16)
        cf = jnp.zeros((Nb, H), F32)
        hb = jnp.zeros((Nb, H), BF16)
        cb = jnp.zeros((Nb, H), F32)
        for k in range(T):
            tb = T - 1 - k
            gf = uf_ref[k] + jnp.dot(hf, whf, preferred_element_type=F32)
            gb = ub_ref[tb] + jnp.dot(hb, whb, preferred_element_type=F32)
            cf = (_sig(gf[:, 1 * H:2 * H]) * cf
                  + _sig(gf[:, 0 * H:1 * H]) * jnp.tanh(gf[:, 2 * H:3 * H]))
            cb = (_sig(gb[:, 1 * H:2 * H]) * cb
                  + _sig(gb[:, 0 * H:1 * H]) * jnp.tanh(gb[:, 2 * H:3 * H]))
            hf = (_sig(gf[:, 3 * H:4 * H]) * jnp.tanh(cf)).astype(BF16)
            hb = (_sig(gb[:, 3 * H:4 * H]) * jnp.tanh(cb)).astype(BF16)
            hsf_ref[k] = hf
            hsb_ref[tb] = hb

    def lin(wf, wb, bias):
        return (jnp.dot(hsf_ref[...].reshape(T * Nb, H), wf[...],
                        preferred_element_type=F32)
                + jnp.dot(hsb_ref[...].reshape(T * Nb, H), wb[...],
                          preferred_element_type=F32)
                + bias[...])

    z1 = zt_ref[...].reshape(T * Nb, C)
    bilstm(z1, w1fih, w1fhh, b1f, w1bih, w1bhh, b1b)
    rec1 = lin(lin1f, lin1b, lin1bias)
    z2_ref[...] = rec1.astype(BF16).reshape(T, Nb, H)

    bilstm(z2_ref[...].reshape(T * Nb, H), w2fih, w2fhh, b2f, w2bih, w2bhh, b2b)
    rec2 = lin(lin2f, lin2b, lin2bias)

    sc = jnp.dot(rec2.astype(BF16), outw[...], preferred_element_type=F32) + outb[...]
    sc3 = sc.reshape(T, Nb, sc.shape[-1])
    for t in range(T):
        scores_ref[:, t, :] = sc3[t]

    deltas_ref[...] = (jnp.dot(y_ref[...].astype(BF16), bboxw[...],
                               preferred_element_type=F32) + bboxb[...])


def kernel(x, y,
           l1_f_Wih, l1_f_Whh, l1_f_bih, l1_f_bhh,
           l1_b_Wih, l1_b_Whh, l1_b_bih, l1_b_bhh,
           l1_lin_W, l1_lin_b,
           l2_f_Wih, l2_f_Whh, l2_f_bih, l2_f_bhh,
           l2_b_Wih, l2_b_Whh, l2_b_bih, l2_b_bhh,
           l2_lin_W, l2_lin_b,
           out_W, out_b, bbox_W, bbox_b):
    N, C, Hh, Ww = x.shape
    T = Ww
    H = l1_f_Whh.shape[1]
    G = 4 * H
    NC = out_W.shape[0]
    FC = y.shape[1]
    Nb = 200
    grid = N // Nb

    zt = _avg_pool(x, Hh, Ww)

    def prep(wih, whh, bih, bhh):
        return (wih.T.astype(BF16), whh.T.astype(BF16),
                (bih + bhh).reshape(1, G))

    w1fih, w1fhh, b1f = prep(l1_f_Wih, l1_f_Whh, l1_f_bih, l1_f_bhh)
    w1bih, w1bhh, b1b = prep(l1_b_Wih, l1_b_Whh, l1_b_bih, l1_b_bhh)
    w2fih, w2fhh, b2f = prep(l2_f_Wih, l2_f_Whh, l2_f_bih, l2_f_bhh)
    w2bih, w2bhh, b2b = prep(l2_b_Wih, l2_b_Whh, l2_b_bih, l2_b_bhh)
    lin1f = l1_lin_W[:, :H].T.astype(BF16)
    lin1b = l1_lin_W[:, H:].T.astype(BF16)
    lin2f = l2_lin_W[:, :H].T.astype(BF16)
    lin2b = l2_lin_W[:, H:].T.astype(BF16)
    lin1bias = l1_lin_b.reshape(1, H)
    lin2bias = l2_lin_b.reshape(1, H)
    outw = out_W.T.astype(BF16)
    outb = out_b.reshape(1, NC)
    bboxw = bbox_W.T.astype(BF16)
    bboxb = bbox_b.reshape(1, 4)

    full = lambda shape: pl.BlockSpec(shape, lambda i: tuple(0 for _ in shape))
    in_specs = [
        pl.BlockSpec((T, Nb, C), lambda i: (0, i, 0)),
        pl.BlockSpec((Nb, FC), lambda i: (i, 0)),
        full((C, G)), full((H, G)), full((1, G)),
        full((C, G)), full((H, G)), full((1, G)),
        full((H, G)), full((H, G)), full((1, G)),
        full((H, G)), full((H, G)), full((1, G)),
        full((H, H)), full((H, H)), full((1, H)),
        full((H, H)), full((H, H)), full((1, H)),
        full((H, NC)), full((1, NC)),
        full((FC, 4)), full((1, 4)),
    ]
    out_specs = [
        pl.BlockSpec((Nb, T, NC), lambda i: (i, 0, 0)),
        pl.BlockSpec((Nb, 4), lambda i: (i, 0)),
    ]
    out_shape = [
        jax.ShapeDtypeStruct((N, T, NC), F32),
        jax.ShapeDtypeStruct((N, 4), F32),
    ]
    scratch_shapes = [
        pltpu.VMEM((T, Nb, G), F32),
        pltpu.VMEM((T, Nb, G), F32),
        pltpu.VMEM((T, Nb, H), BF16),
        pltpu.VMEM((T, Nb, H), BF16),
        pltpu.VMEM((T, Nb, H), BF16),
    ]

    scores, deltas = pl.pallas_call(
        functools.partial(_main_body, T, Nb, C, H),
        grid=(grid,),
        in_specs=in_specs,
        out_specs=out_specs,
        out_shape=out_shape,
        scratch_shapes=scratch_shapes,
    )(zt, y,
      w1fih, w1fhh, b1f, w1bih, w1bhh, b1b,
      w2fih, w2fhh, b2f, w2bih, w2bhh, b2b,
      lin1f, lin1b, lin1bias, lin2f, lin2b, lin2bias,
      outw, outb, bboxw, bboxb)

    return (scores, deltas)
